# baseline (speedup 1.0000x reference)
import functools

import jax
import jax.numpy as jnp
from jax import lax
from jax.experimental import pallas as pl
from jax.experimental.pallas import tpu as pltpu
from jax.experimental.pallas import tpu_sc as plsc


def _make_gather(V, D, B, NC, NS):
  NW = NC * NS
  assert D % 16 == 0 and B % (8 * NW) == 0
  b_per_w = B // NW
  mesh = plsc.VectorSubcoreMesh(core_axis_name="c", subcore_axis_name="s")

  @functools.partial(
      pl.kernel,
      out_type=jax.ShapeDtypeStruct((B, D), jnp.float32),
      mesh=mesh,
      scratch_types=[
          pltpu.VMEM((b_per_w,), jnp.int32),
          pltpu.VMEM((b_per_w, D), jnp.float32),
          pltpu.SemaphoreType.DMA,
      ],
  )
  def gather(idx_hbm, table_hbm, out_hbm, idx_v, rows_v, sem):
    wid = lax.axis_index("s") * NC + lax.axis_index("c")
    base = wid * b_per_w
    pltpu.sync_copy(idx_hbm.at[pl.ds(base, b_per_w)], idx_v)
    pltpu.async_copy(table_hbm.at[idx_v], rows_v, sem).wait()
    pltpu.sync_copy(rows_v, out_hbm.at[pl.ds(base, b_per_w)])

  return gather


def _head_body(x_ref, w_ref, b_ref, o_ref):
  x = x_ref[...]
  w = w_ref[...]
  logits = lax.dot_general(
      w, x, (((1,), (1,)), ((), ())), preferred_element_type=jnp.float32)
  logits = logits + b_ref[...]
  m = jnp.max(logits, axis=0, keepdims=True)
  s = logits - m
  lse = jnp.log(jnp.sum(jnp.exp(s), axis=0, keepdims=True))
  o_ref[...] = s - lse


def _head(embeds, W, b2, BM):
  B, D = embeds.shape
  NY = W.shape[0]
  return pl.pallas_call(
      _head_body,
      grid=(B // BM,),
      in_specs=[
          pl.BlockSpec((BM, D), lambda i: (i, 0)),
          pl.BlockSpec((NY, D), lambda i: (0, 0)),
          pl.BlockSpec((NY, 1), lambda i: (0, 0)),
      ],
      out_specs=pl.BlockSpec((NY, BM), lambda i: (0, i)),
      out_shape=jax.ShapeDtypeStruct((NY, B), jnp.float32),
  )(embeds, W, b2)


def kernel(text, emb, W, b):
  B, = text.shape
  V, D = emb.shape
  NY = W.shape[0]
  info = plsc.get_sparse_core_info()
  gather = _make_gather(V, D, B, info.num_cores, info.num_subcores)
  embeds = gather(text.astype(jnp.int32), emb)
  outT = _head(embeds, W, b.reshape(NY, 1), BM=4096)
  return outT.T


# speedup vs baseline: 1.0298x; 1.0298x over previous
import functools

import jax
import jax.numpy as jnp
from jax import lax
from jax.experimental import pallas as pl
from jax.experimental.pallas import tpu as pltpu
from jax.experimental.pallas import tpu_sc as plsc


def _make_gather(V, D, B, NC, NS):
  NW = NC * NS
  assert D % 16 == 0 and B % (8 * NW) == 0
  b_per_w = B // NW
  mesh = plsc.VectorSubcoreMesh(core_axis_name="c", subcore_axis_name="s")

  @functools.partial(
      pl.kernel,
      out_type=jax.ShapeDtypeStruct((B, D), jnp.float32),
      mesh=mesh,
      scratch_types=[
          pltpu.VMEM((b_per_w,), jnp.int32),
          pltpu.VMEM((b_per_w, D), jnp.float32),
          pltpu.SemaphoreType.DMA,
      ],
  )
  def gather(idx_hbm, table_hbm, out_hbm, idx_v, rows_v, sem):
    wid = lax.axis_index("s") * NC + lax.axis_index("c")
    base = wid * b_per_w
    pltpu.sync_copy(idx_hbm.at[pl.ds(base, b_per_w)], idx_v)
    pltpu.async_copy(table_hbm.at[idx_v], rows_v, sem).wait()
    pltpu.sync_copy(rows_v, out_hbm.at[pl.ds(base, b_per_w)])

  return gather


def _head_body(x_ref, w_ref, b_ref, o_ref):
  x = x_ref[...]
  w = w_ref[...]
  logits = lax.dot_general(
      w, x, (((1,), (1,)), ((), ())), preferred_element_type=jnp.float32)
  logits = logits + b_ref[...]
  m = jnp.max(logits, axis=0, keepdims=True)
  s = logits - m
  lse = jnp.log(jnp.sum(jnp.exp(s), axis=0, keepdims=True))
  o_ref[...] = s - lse


def _head(embeds, W, b2, BM):
  B, D = embeds.shape
  NY = W.shape[0]
  return pl.pallas_call(
      _head_body,
      grid=(B // BM,),
      in_specs=[
          pl.BlockSpec((BM, D), lambda i: (i, 0)),
          pl.BlockSpec((NY, D), lambda i: (0, 0)),
          pl.BlockSpec((NY, 1), lambda i: (0, 0)),
      ],
      out_specs=pl.BlockSpec((NY, BM), lambda i: (0, i)),
      out_shape=jax.ShapeDtypeStruct((NY, B), jnp.float32),
  )(embeds, W, b2)


def kernel(text, emb, W, b):
  B, = text.shape
  V, D = emb.shape
  NY = W.shape[0]
  info = plsc.get_sparse_core_info()
  gather = _make_gather(V, D, B, info.num_cores, info.num_subcores)
  embeds = gather(text.astype(jnp.int32), emb)
  outT = _head(embeds, W, b.reshape(NY, 1), BM=1024)
  return outT.T


# speedup vs baseline: 1.0519x; 1.0214x over previous
import functools

import jax
import jax.numpy as jnp
from jax import lax
from jax.experimental import pallas as pl
from jax.experimental.pallas import tpu as pltpu
from jax.experimental.pallas import tpu_sc as plsc


def _make_gather(V, D, B, NC, NS):
  NW = NC * NS
  assert D % 16 == 0 and B % (8 * NW) == 0
  b_per_w = B // NW
  mesh = plsc.VectorSubcoreMesh(core_axis_name="c", subcore_axis_name="s")

  @functools.partial(
      pl.kernel,
      out_type=jax.ShapeDtypeStruct((B, D), jnp.float32),
      mesh=mesh,
      scratch_types=[
          pltpu.VMEM((b_per_w,), jnp.int32),
          pltpu.VMEM((b_per_w, D), jnp.float32),
          pltpu.SemaphoreType.DMA,
      ],
  )
  def gather(idx_hbm, table_hbm, out_hbm, idx_v, rows_v, sem):
    wid = lax.axis_index("s") * NC + lax.axis_index("c")
    base = wid * b_per_w
    pltpu.sync_copy(idx_hbm.at[pl.ds(base, b_per_w)], idx_v)
    pltpu.async_copy(table_hbm.at[idx_v], rows_v, sem).wait()
    pltpu.sync_copy(rows_v, out_hbm.at[pl.ds(base, b_per_w)])

  return gather


def _head_body(x_ref, w_ref, b_ref, o_ref):
  x = x_ref[...]
  w = w_ref[...]
  logits = lax.dot_general(
      w, x, (((1,), (1,)), ((), ())), preferred_element_type=jnp.float32)
  logits = logits + b_ref[...]
  m = jnp.max(logits, axis=0, keepdims=True)
  s = logits - m
  lse = jnp.log(jnp.sum(jnp.exp(s), axis=0, keepdims=True))
  o_ref[...] = s - lse


def _head(embeds, W, b2, BM):
  B, D = embeds.shape
  NY = W.shape[0]
  return pl.pallas_call(
      _head_body,
      grid=(B // BM,),
      in_specs=[
          pl.BlockSpec((BM, D), lambda i: (i, 0)),
          pl.BlockSpec((NY, D), lambda i: (0, 0)),
          pl.BlockSpec((NY, 1), lambda i: (0, 0)),
      ],
      out_specs=pl.BlockSpec((NY, BM), lambda i: (0, i)),
      out_shape=jax.ShapeDtypeStruct((NY, B), jnp.float32),
  )(embeds, W, b2)


def kernel(text, emb, W, b):
  B, = text.shape
  V, D = emb.shape
  NY = W.shape[0]
  info = plsc.get_sparse_core_info()
  gather = _make_gather(V, D, B, info.num_cores, info.num_subcores)
  embeds = gather(text.astype(jnp.int32), emb)
  outT = _head(embeds, W, b.reshape(NY, 1), BM=2048)
  return outT.T
